# baseline (speedup 1.0000x reference)
import functools

import jax
import jax.numpy as jnp
from jax import lax
from jax.experimental import pallas as pl
from jax.experimental.pallas import tpu as pltpu
from jax.experimental.pallas import tpu_sc as plsc

N = 10000
E = 320000
F = 128
H = 128
C_OUT = 16
B_SEG = 64

NC = 2
NS = 16
NW = NC * NS
CH = 128
NCHUNK = 79
EW = CH * NCHUNK
EPAD = EW * NW
CHUNKS_C0 = 52
CHUNKS_C1 = 106
NPAD = 10240
RPT = NPAD // NS
NROW = NPAD // CH

RB = 1000
GRID = N // RB


_MESH = plsc.VectorSubcoreMesh(core_axis_name="c", subcore_axis_name="s")


@functools.partial(
    pl.kernel,
    out_type=jax.ShapeDtypeStruct((NC, NPAD, H), jnp.float32),
    mesh=_MESH,
    scratch_types=[
        pltpu.VMEM((CH,), jnp.int32),
        pltpu.VMEM((CH, H), jnp.float32),
        pltpu.VMEM_SHARED((NPAD, H), jnp.float32),
    ],
)
def _deg_kernel(dst_hbm, ones_hbm, zeros_hbm, out_hbm, dst_v, ones_v, acc):
    c = lax.axis_index("c")
    s = lax.axis_index("s")
    w = c * NS + s
    pltpu.sync_copy(zeros_hbm.at[pl.ds(s * RPT, RPT)], acc.at[pl.ds(s * RPT, RPT)])
    pltpu.sync_copy(ones_hbm, ones_v)
    plsc.subcore_barrier()
    base = w * EW

    @pl.loop(0, NCHUNK)
    def _(j):
        off = pl.multiple_of(base + j * CH, 8)
        pltpu.sync_copy(dst_hbm.at[pl.ds(off, CH)], dst_v)
        pltpu.sync_copy(ones_v, acc.at[dst_v], add=True)

    plsc.subcore_barrier()
    pltpu.sync_copy(acc.at[pl.ds(s * RPT, RPT)], out_hbm.at[c].at[pl.ds(s * RPT, RPT)])


@functools.partial(
    pl.kernel,
    out_type=jax.ShapeDtypeStruct((NC, NPAD, H), jnp.float32),
    mesh=_MESH,
    scratch_types=[
        pltpu.VMEM((CH,), jnp.int32),
        pltpu.VMEM((CH,), jnp.int32),
        pltpu.VMEM((CH,), jnp.int32),
        pltpu.VMEM((CH,), jnp.int32),
        pltpu.VMEM((CH, H), jnp.float32),
        pltpu.VMEM((CH, H), jnp.float32),
        pltpu.VMEM_SHARED((NPAD, H), jnp.float32),
        pltpu.SemaphoreType.DMA,
        pltpu.SemaphoreType.DMA,
    ],
)
def _msg_kernel(g_hbm, src_hbm, dst_hbm, zeros_hbm, out_hbm,
                src_v0, src_v1, dst_v0, dst_v1, rows0, rows1, acc, sem0, sem1):
    c = lax.axis_index("c")
    s = lax.axis_index("s")
    pltpu.sync_copy(zeros_hbm.at[pl.ds(s * RPT, RPT)], acc.at[pl.ds(s * RPT, RPT)])
    plsc.subcore_barrier()
    nch = CHUNKS_C0 + c * (CHUNKS_C1 - CHUNKS_C0)
    base = (c * NS * CHUNKS_C0 + s * nch) * CH

    @pl.loop(0, CHUNKS_C0 // 2 + c * (CHUNKS_C1 - CHUNKS_C0) // 2)
    def _(k):
        off0 = pl.multiple_of(base + (2 * k) * CH, 8)
        off1 = pl.multiple_of(base + (2 * k + 1) * CH, 8)
        pltpu.sync_copy(src_hbm.at[pl.ds(off0, CH)], src_v0)
        d0 = pltpu.async_copy(g_hbm.at[src_v0], rows0, sem0)
        pltpu.sync_copy(src_hbm.at[pl.ds(off1, CH)], src_v1)
        d1 = pltpu.async_copy(g_hbm.at[src_v1], rows1, sem1)
        pltpu.sync_copy(dst_hbm.at[pl.ds(off0, CH)], dst_v0)
        pltpu.sync_copy(dst_hbm.at[pl.ds(off1, CH)], dst_v1)
        d0.wait()
        pltpu.sync_copy(rows0, acc.at[dst_v0], add=True)
        d1.wait()
        pltpu.sync_copy(rows1, acc.at[dst_v1], add=True)

    plsc.subcore_barrier()
    pltpu.sync_copy(acc.at[pl.ds(s * RPT, RPT)], out_hbm.at[c].at[pl.ds(s * RPT, RPT)])


def _tc_first(x, w_in, d0, d1):
    def body(x_ref, w_ref, d0_ref, d1_ref, g_ref, dinv_ref):
        deg = d0_ref[...] + d1_ref[...] + 1.0
        dinv = lax.rsqrt(deg)
        hw = jnp.dot(x_ref[...], w_ref[...], preferred_element_type=jnp.float32)
        g_ref[...] = dinv * hw
        dinv_ref[...] = dinv

    return pl.pallas_call(
        body,
        grid=(GRID,),
        in_specs=[
            pl.BlockSpec((RB, F), lambda i: (i, 0)),
            pl.BlockSpec((F, H), lambda i: (0, 0)),
            pl.BlockSpec((RB, 1), lambda i: (i, 0)),
            pl.BlockSpec((RB, 1), lambda i: (i, 0)),
        ],
        out_specs=[
            pl.BlockSpec((RB, H), lambda i: (i, 0)),
            pl.BlockSpec((RB, 1), lambda i: (i, 0)),
        ],
        out_shape=[
            jax.ShapeDtypeStruct((N, H), jnp.float32),
            jax.ShapeDtypeStruct((N, 1), jnp.float32),
        ],
    )(x, w_in, d0, d1)


def _tc_mid(m0, m1, g_prev, dinv, w, b):
    def body(m0_ref, m1_ref, g_ref, dinv_ref, w_ref, b_ref, out_ref):
        dinv = dinv_ref[...]
        t = dinv * (m0_ref[...] + m1_ref[...] + g_ref[...]) + b_ref[...]
        t = jnp.maximum(t, 0.0)
        out_ref[...] = dinv * jnp.dot(t, w_ref[...],
                                      preferred_element_type=jnp.float32)

    return pl.pallas_call(
        body,
        grid=(GRID,),
        in_specs=[
            pl.BlockSpec((RB, H), lambda i: (i, 0)),
            pl.BlockSpec((RB, H), lambda i: (i, 0)),
            pl.BlockSpec((RB, H), lambda i: (i, 0)),
            pl.BlockSpec((RB, 1), lambda i: (i, 0)),
            pl.BlockSpec((H, H), lambda i: (0, 0)),
            pl.BlockSpec((1, H), lambda i: (0, 0)),
        ],
        out_specs=pl.BlockSpec((RB, H), lambda i: (i, 0)),
        out_shape=jax.ShapeDtypeStruct((N, H), jnp.float32),
    )(m0, m1, g_prev, dinv, w, b)


def _tc_final(m0, m1, g_prev, dinv, b, batch2d, w_mlp, b_mlp):
    def body(m0_ref, m1_ref, g_ref, dinv_ref, b_ref, batch_ref, wm_ref,
             bm_ref, out_ref, s_acc, c_acc):
        i = pl.program_id(0)

        @pl.when(i == 0)
        def _():
            s_acc[...] = jnp.zeros_like(s_acc)
            c_acc[...] = jnp.zeros_like(c_acc)

        t = dinv_ref[...] * (m0_ref[...] + m1_ref[...] + g_ref[...]) + b_ref[...]
        t = jnp.maximum(t, 0.0)
        seg = lax.broadcasted_iota(jnp.int32, (RB, B_SEG), 1)
        onehot = (batch_ref[...] == seg).astype(jnp.float32)
        dn = (((0,), (0,)), ((), ()))
        s_acc[...] += lax.dot_general(onehot, t, dn,
                                      preferred_element_type=jnp.float32)
        c_acc[...] += lax.dot_general(onehot, jnp.ones_like(t), dn,
                                      preferred_element_type=jnp.float32)

        @pl.when(i == GRID - 1)
        def _():
            pooled = s_acc[...] / jnp.maximum(c_acc[...], 1.0)
            out_ref[...] = jnp.dot(pooled, wm_ref[...],
                                   preferred_element_type=jnp.float32) + bm_ref[...]

    return pl.pallas_call(
        body,
        grid=(GRID,),
        in_specs=[
            pl.BlockSpec((RB, H), lambda i: (i, 0)),
            pl.BlockSpec((RB, H), lambda i: (i, 0)),
            pl.BlockSpec((RB, H), lambda i: (i, 0)),
            pl.BlockSpec((RB, 1), lambda i: (i, 0)),
            pl.BlockSpec((1, H), lambda i: (0, 0)),
            pl.BlockSpec((RB, 1), lambda i: (i, 0)),
            pl.BlockSpec((H, C_OUT), lambda i: (0, 0)),
            pl.BlockSpec((1, C_OUT), lambda i: (0, 0)),
        ],
        out_specs=pl.BlockSpec((B_SEG, C_OUT), lambda i: (0, 0)),
        out_shape=jax.ShapeDtypeStruct((B_SEG, C_OUT), jnp.float32),
        scratch_shapes=[
            pltpu.VMEM((B_SEG, H), jnp.float32),
            pltpu.VMEM((B_SEG, H), jnp.float32),
        ],
    )(m0, m1, g_prev, dinv, b, batch2d, w_mlp, b_mlp)


def kernel(x, edge_index, batch, W_in, b_in, W_mid, b_mid, W_mlp, b_mlp):
    src = edge_index[0]
    dst = edge_index[1]
    pad_e = EPAD - E
    src_pad = jnp.concatenate([src, jnp.zeros((pad_e,), jnp.int32)])
    dst_pad = jnp.concatenate([dst, jnp.full((pad_e,), N, jnp.int32)])
    zeros_msg = jnp.zeros((NPAD, H), jnp.float32)

    ones_rows = jnp.ones((CH, H), jnp.float32)
    deg = _deg_kernel(dst_pad, ones_rows, zeros_msg)
    g, dinv = _tc_first(x, W_in, deg[0][:N, :1], deg[1][:N, :1])

    b_in2 = b_in.reshape(1, H)
    b_mid2 = b_mid.reshape(1, H)
    b_mlp2 = b_mlp.reshape(1, C_OUT)

    for layer in range(3):
        m = _msg_kernel(g, src_pad, dst_pad, zeros_msg)
        bias = b_in2 if layer == 0 else b_mid2
        g = _tc_mid(m[0], m[1], g, dinv, W_mid, bias)

    m = _msg_kernel(g, src_pad, dst_pad, zeros_msg)
    batch2d = batch.reshape(N, 1)
    return _tc_final(m[0], m[1], g, dinv, b_mid2, batch2d, W_mlp, b_mlp2)


# speedup vs baseline: 1.2403x; 1.2403x over previous
import functools

import jax
import jax.numpy as jnp
from jax import lax
from jax.experimental import pallas as pl
from jax.experimental.pallas import tpu as pltpu
from jax.experimental.pallas import tpu_sc as plsc

N = 10000
E = 320000
F = 128
H = 128
C_OUT = 16
B_SEG = 64

NC = 2
NS = 16
NW = NC * NS
CH = 128
NCHUNK = 79
EW = CH * NCHUNK
EPAD = EW * NW
CHUNKS_C0 = 106
CHUNKS_C1 = 52
NPAD = 10240
RPT = NPAD // NS
NROW = NPAD // CH

RB = 1000
GRID = N // RB


_MESH = plsc.VectorSubcoreMesh(core_axis_name="c", subcore_axis_name="s")


@functools.partial(
    pl.kernel,
    out_type=jax.ShapeDtypeStruct((NC, NPAD, H), jnp.float32),
    mesh=_MESH,
    scratch_types=[
        pltpu.VMEM((CH,), jnp.int32),
        pltpu.VMEM((CH, H), jnp.float32),
        pltpu.VMEM_SHARED((NPAD, H), jnp.float32),
    ],
)
def _deg_kernel(dst_hbm, ones_hbm, zeros_hbm, out_hbm, dst_v, ones_v, acc):
    c = lax.axis_index("c")
    s = lax.axis_index("s")
    w = c * NS + s
    pltpu.sync_copy(zeros_hbm.at[pl.ds(s * RPT, RPT)], acc.at[pl.ds(s * RPT, RPT)])
    pltpu.sync_copy(ones_hbm, ones_v)
    plsc.subcore_barrier()
    base = w * EW

    @pl.loop(0, NCHUNK)
    def _(j):
        off = pl.multiple_of(base + j * CH, 8)
        pltpu.sync_copy(dst_hbm.at[pl.ds(off, CH)], dst_v)
        pltpu.sync_copy(ones_v, acc.at[dst_v], add=True)

    plsc.subcore_barrier()
    pltpu.sync_copy(acc.at[pl.ds(s * RPT, RPT)], out_hbm.at[c].at[pl.ds(s * RPT, RPT)])


@functools.partial(
    pl.kernel,
    out_type=jax.ShapeDtypeStruct((NC, NPAD, H), jnp.float32),
    mesh=_MESH,
    scratch_types=[
        pltpu.VMEM((CH,), jnp.int32),
        pltpu.VMEM((CH,), jnp.int32),
        pltpu.VMEM((CH,), jnp.int32),
        pltpu.VMEM((CH,), jnp.int32),
        pltpu.VMEM((CH, H), jnp.float32),
        pltpu.VMEM((CH, H), jnp.float32),
        pltpu.VMEM_SHARED((NPAD, H), jnp.float32),
        pltpu.SemaphoreType.DMA,
        pltpu.SemaphoreType.DMA,
    ],
)
def _msg_kernel(g_hbm, src_hbm, dst_hbm, zeros_hbm, out_hbm,
                src_v0, src_v1, dst_v0, dst_v1, rows0, rows1, acc, sem0, sem1):
    c = lax.axis_index("c")
    s = lax.axis_index("s")
    pltpu.sync_copy(zeros_hbm.at[pl.ds(s * RPT, RPT)], acc.at[pl.ds(s * RPT, RPT)])
    plsc.subcore_barrier()
    nch = CHUNKS_C0 + c * (CHUNKS_C1 - CHUNKS_C0)
    base = (c * NS * CHUNKS_C0 + s * nch) * CH

    @pl.loop(0, CHUNKS_C0 // 2 + c * (CHUNKS_C1 - CHUNKS_C0) // 2)
    def _(k):
        off0 = pl.multiple_of(base + (2 * k) * CH, 8)
        off1 = pl.multiple_of(base + (2 * k + 1) * CH, 8)
        pltpu.sync_copy(src_hbm.at[pl.ds(off0, CH)], src_v0)
        d0 = pltpu.async_copy(g_hbm.at[src_v0], rows0, sem0)
        pltpu.sync_copy(src_hbm.at[pl.ds(off1, CH)], src_v1)
        d1 = pltpu.async_copy(g_hbm.at[src_v1], rows1, sem1)
        pltpu.sync_copy(dst_hbm.at[pl.ds(off0, CH)], dst_v0)
        pltpu.sync_copy(dst_hbm.at[pl.ds(off1, CH)], dst_v1)
        d0.wait()
        pltpu.sync_copy(rows0, acc.at[dst_v0], add=True)
        d1.wait()
        pltpu.sync_copy(rows1, acc.at[dst_v1], add=True)

    plsc.subcore_barrier()
    pltpu.sync_copy(acc.at[pl.ds(s * RPT, RPT)], out_hbm.at[c].at[pl.ds(s * RPT, RPT)])


def _tc_first(x, w_in, d0, d1):
    def body(x_ref, w_ref, d0_ref, d1_ref, g_ref, dinv_ref):
        deg = d0_ref[...] + d1_ref[...] + 1.0
        dinv = lax.rsqrt(deg)
        hw = jnp.dot(x_ref[...], w_ref[...], preferred_element_type=jnp.float32)
        g_ref[...] = dinv * hw
        dinv_ref[...] = dinv

    return pl.pallas_call(
        body,
        grid=(GRID,),
        in_specs=[
            pl.BlockSpec((RB, F), lambda i: (i, 0)),
            pl.BlockSpec((F, H), lambda i: (0, 0)),
            pl.BlockSpec((RB, 1), lambda i: (i, 0)),
            pl.BlockSpec((RB, 1), lambda i: (i, 0)),
        ],
        out_specs=[
            pl.BlockSpec((RB, H), lambda i: (i, 0)),
            pl.BlockSpec((RB, 1), lambda i: (i, 0)),
        ],
        out_shape=[
            jax.ShapeDtypeStruct((N, H), jnp.float32),
            jax.ShapeDtypeStruct((N, 1), jnp.float32),
        ],
    )(x, w_in, d0, d1)


def _tc_mid(m0, m1, g_prev, dinv, w, b):
    def body(m0_ref, m1_ref, g_ref, dinv_ref, w_ref, b_ref, out_ref):
        dinv = dinv_ref[...]
        t = dinv * (m0_ref[...] + m1_ref[...] + g_ref[...]) + b_ref[...]
        t = jnp.maximum(t, 0.0)
        out_ref[...] = dinv * jnp.dot(t, w_ref[...],
                                      preferred_element_type=jnp.float32)

    return pl.pallas_call(
        body,
        grid=(GRID,),
        in_specs=[
            pl.BlockSpec((RB, H), lambda i: (i, 0)),
            pl.BlockSpec((RB, H), lambda i: (i, 0)),
            pl.BlockSpec((RB, H), lambda i: (i, 0)),
            pl.BlockSpec((RB, 1), lambda i: (i, 0)),
            pl.BlockSpec((H, H), lambda i: (0, 0)),
            pl.BlockSpec((1, H), lambda i: (0, 0)),
        ],
        out_specs=pl.BlockSpec((RB, H), lambda i: (i, 0)),
        out_shape=jax.ShapeDtypeStruct((N, H), jnp.float32),
    )(m0, m1, g_prev, dinv, w, b)


def _tc_final(m0, m1, g_prev, dinv, b, batch2d, w_mlp, b_mlp):
    def body(m0_ref, m1_ref, g_ref, dinv_ref, b_ref, batch_ref, wm_ref,
             bm_ref, out_ref, s_acc, c_acc):
        i = pl.program_id(0)

        @pl.when(i == 0)
        def _():
            s_acc[...] = jnp.zeros_like(s_acc)
            c_acc[...] = jnp.zeros_like(c_acc)

        t = dinv_ref[...] * (m0_ref[...] + m1_ref[...] + g_ref[...]) + b_ref[...]
        t = jnp.maximum(t, 0.0)
        seg = lax.broadcasted_iota(jnp.int32, (RB, B_SEG), 1)
        onehot = (batch_ref[...] == seg).astype(jnp.float32)
        dn = (((0,), (0,)), ((), ()))
        s_acc[...] += lax.dot_general(onehot, t, dn,
                                      preferred_element_type=jnp.float32)
        c_acc[...] += lax.dot_general(onehot, jnp.ones_like(t), dn,
                                      preferred_element_type=jnp.float32)

        @pl.when(i == GRID - 1)
        def _():
            pooled = s_acc[...] / jnp.maximum(c_acc[...], 1.0)
            out_ref[...] = jnp.dot(pooled, wm_ref[...],
                                   preferred_element_type=jnp.float32) + bm_ref[...]

    return pl.pallas_call(
        body,
        grid=(GRID,),
        in_specs=[
            pl.BlockSpec((RB, H), lambda i: (i, 0)),
            pl.BlockSpec((RB, H), lambda i: (i, 0)),
            pl.BlockSpec((RB, H), lambda i: (i, 0)),
            pl.BlockSpec((RB, 1), lambda i: (i, 0)),
            pl.BlockSpec((1, H), lambda i: (0, 0)),
            pl.BlockSpec((RB, 1), lambda i: (i, 0)),
            pl.BlockSpec((H, C_OUT), lambda i: (0, 0)),
            pl.BlockSpec((1, C_OUT), lambda i: (0, 0)),
        ],
        out_specs=pl.BlockSpec((B_SEG, C_OUT), lambda i: (0, 0)),
        out_shape=jax.ShapeDtypeStruct((B_SEG, C_OUT), jnp.float32),
        scratch_shapes=[
            pltpu.VMEM((B_SEG, H), jnp.float32),
            pltpu.VMEM((B_SEG, H), jnp.float32),
        ],
    )(m0, m1, g_prev, dinv, b, batch2d, w_mlp, b_mlp)


def kernel(x, edge_index, batch, W_in, b_in, W_mid, b_mid, W_mlp, b_mlp):
    src = edge_index[0]
    dst = edge_index[1]
    pad_e = EPAD - E
    src_pad = jnp.concatenate([src, jnp.zeros((pad_e,), jnp.int32)])
    dst_pad = jnp.concatenate([dst, jnp.full((pad_e,), N, jnp.int32)])
    zeros_msg = jnp.zeros((NPAD, H), jnp.float32)

    ones_rows = jnp.ones((CH, H), jnp.float32)
    deg = _deg_kernel(dst_pad, ones_rows, zeros_msg)
    g, dinv = _tc_first(x, W_in, deg[0][:N, :1], deg[1][:N, :1])

    b_in2 = b_in.reshape(1, H)
    b_mid2 = b_mid.reshape(1, H)
    b_mlp2 = b_mlp.reshape(1, C_OUT)

    for layer in range(3):
        m = _msg_kernel(g, src_pad, dst_pad, zeros_msg)
        bias = b_in2 if layer == 0 else b_mid2
        g = _tc_mid(m[0], m[1], g, dinv, W_mid, bias)

    m = _msg_kernel(g, src_pad, dst_pad, zeros_msg)
    batch2d = batch.reshape(N, 1)
    return _tc_final(m[0], m[1], g, dinv, b_mid2, batch2d, W_mlp, b_mlp2)
